# Initial kernel scaffold; baseline (speedup 1.0000x reference)
import jax
import jax.numpy as jnp
from jax.experimental import pallas as pl


def kernel(ind_2, prop, inter):
    raise NotImplementedError("write your pallas kernel here")


# speedup vs baseline: 6.3442x; 6.3442x over previous
import functools

import jax
import jax.numpy as jnp
from jax import lax
from jax.experimental import pallas as pl
from jax.experimental.pallas import tpu as pltpu
from jax.experimental.pallas import tpu_sc as plsc

NC = 2
NS = 16
NW = NC * NS
LANES = 16
IDXB = 128
CH_I = 8
CH_P = CH_I * IDXB
RW = 200


def _sc_partials(idx2d, inter, *, n_atoms, n_pairs):
    n_chunks = n_pairs // CH_P
    n_rchunks = n_atoms // RW

    mesh = plsc.VectorSubcoreMesh(core_axis_name="c", subcore_axis_name="s")

    @functools.partial(
        pl.kernel,
        out_type=jax.ShapeDtypeStruct((NC, n_atoms, LANES), jnp.float32),
        mesh=mesh,
        scratch_types=[
            pltpu.VMEM((CH_I, IDXB), jnp.int32),
            pltpu.VMEM((CH_P, LANES), jnp.float32),
            pltpu.VMEM_SHARED((n_atoms, LANES), jnp.float32),
            pltpu.SemaphoreType.DMA,
        ],
        compiler_params=pltpu.CompilerParams(use_tc_tiling_on_sc=False),
    )
    def body(idx_hbm, inter_hbm, out_hbm, idxv, rows, acc, sem):
        c = lax.axis_index("c")
        s = lax.axis_index("s")
        w = s * NC + c

        def zero_row(i, _):
            rows[i] = jnp.zeros((LANES,), jnp.float32)
            return 0
        lax.fori_loop(0, RW, zero_row, 0)

        n_z = (n_rchunks - s + NS - 1) // NS

        def zero_chunk(z, _):
            zc = s + z * NS
            pltpu.sync_copy(rows.at[pl.ds(0, RW)], acc.at[pl.ds(zc * RW, RW)])
            return 0

        lax.fori_loop(0, n_z, zero_chunk, 0)
        plsc.subcore_barrier()

        n_k = (n_chunks - w + NW - 1) // NW

        def chunk_body(k, _):
            cid = w + k * NW
            pltpu.sync_copy(idx_hbm.at[pl.ds(cid * CH_I, CH_I)], idxv)
            pltpu.sync_copy(inter_hbm.at[pl.ds(cid * CH_P, CH_P)], rows)
            descs = [
                pltpu.async_copy(rows.at[pl.ds(j * IDXB, IDXB)],
                                 acc.at[idxv.at[j]], sem, add=True)
                for j in range(CH_I)
            ]
            for dsc in descs:
                dsc.wait()
            return 0

        lax.fori_loop(0, n_k, chunk_body, 0)
        plsc.subcore_barrier()

        def dump_chunk(z, _):
            zc = s + z * NS
            pltpu.sync_copy(acc.at[pl.ds(zc * RW, RW)],
                            out_hbm.at[c, pl.ds(zc * RW, RW)])
            return 0

        lax.fori_loop(0, n_z, dump_chunk, 0)

    return body(idx2d, inter)


def _merge_body(p_ref, o_ref):
    o_ref[...] = p_ref[0] + p_ref[1]


def kernel(ind_2, prop, inter):
    n_atoms = prop.shape[0]
    n_pairs, n_inter = inter.shape
    assert n_inter == LANES
    assert n_pairs % CH_P == 0
    assert n_atoms % RW == 0
    assert (n_atoms * LANES) % 128 == 0

    idx2d = ind_2[:, 0].reshape(n_pairs // IDXB, IDXB)
    partials = _sc_partials(idx2d, inter, n_atoms=n_atoms, n_pairs=n_pairs)

    wide = n_atoms * LANES // 128
    pr = partials.reshape(NC, wide, 128)
    merged = pl.pallas_call(
        _merge_body,
        out_shape=jax.ShapeDtypeStruct((wide, 128), jnp.float32),
    )(pr)
    return merged.reshape(n_atoms, LANES)
